# baseline (speedup 1.0000x reference)
import functools

import jax
import jax.numpy as jnp
from jax.experimental import pallas as pl
from jax.experimental.pallas import tpu as pltpu

_N_NODES = 2000
_N_EDGES = 6000
_BATCH = 4
_DIMS = [8, 32, 16, 8, 2]
_HOPS = [3, 3, 3]
_SLOPE = 0.01
_BN_EPS = 1e-5

_FC_IN = _DIMS[-2] * _N_NODES
_FC_OUT = _DIMS[-1] * _N_NODES

_KT = 3200
_NT = 512
_FC_OUT_PAD = 4096


def _fc_body(x_ref, w_ref, b_ref, m_ref, o_ref):
    n = pl.program_id(0)
    k = pl.program_id(1)
    nk = pl.num_programs(1)

    @pl.when(k == 0)
    def _init():
        o_ref[...] = jnp.zeros_like(o_ref)

    o_ref[...] += jnp.dot(x_ref[...], w_ref[...],
                          preferred_element_type=jnp.float32)

    @pl.when(k == nk - 1)
    def _fini():
        b = b_ref[0, pl.ds(n * _NT, _NT)]
        m = m_ref[0, pl.ds(n * _NT, _NT)]
        o_ref[...] = (o_ref[...] + b[None, :]) * m[None, :]


@jax.jit
def _fc_pallas(x2d, fc_w, fc_b, mask_flat):
    pad = _FC_OUT_PAD - _FC_OUT
    b_pad = jnp.pad(fc_b, (0, pad)).reshape(1, _FC_OUT_PAD)
    m_pad = jnp.pad(mask_flat, (0, pad)).reshape(1, _FC_OUT_PAD)
    grid = (_FC_OUT_PAD // _NT, _FC_IN // _KT // 2)
    y = pl.pallas_call(
        _fc_body,
        grid=grid,
        in_specs=[
            pl.BlockSpec((_BATCH, _KT), lambda n, k: (0, k)),
            pl.BlockSpec((_KT, _NT), lambda n, k: (k, n)),
            pl.BlockSpec((1, _FC_OUT_PAD), lambda n, k: (0, 0)),
            pl.BlockSpec((1, _FC_OUT_PAD), lambda n, k: (0, 0)),
        ],
        out_specs=pl.BlockSpec((_BATCH, _NT), lambda n, k: (0, n)),
        out_shape=jax.ShapeDtypeStruct((_BATCH, _FC_OUT_PAD), jnp.float32),
        compiler_params=pltpu.CompilerParams(
            dimension_semantics=("parallel", "arbitrary"),
        ),
    )(x2d, fc_w, b_pad, m_pad)
    return y[:, :_FC_OUT]


def _graph_phase(x, src, dst, ew, conv_weights, conv_biases, bn_gamma, bn_beta):
    deg = jnp.zeros((_N_NODES,), jnp.float32).at[dst].add(ew)
    dinv = jnp.where(deg > 0, 1.0 / jnp.sqrt(deg), 0.0)
    norm = dinv[src] * ew * dinv[dst]

    out = x
    for layer in range(len(_HOPS)):
        h = out
        acc = h @ conv_weights[layer][0]
        for w_hop in conv_weights[layer][1:]:
            msgs = h[:, src, :] * norm[None, :, None]
            h = jnp.zeros_like(h).at[:, dst, :].add(msgs)
            acc = acc + h @ w_hop
        out = acc + conv_biases[layer]
        mu = jnp.mean(out, axis=(0, 2), keepdims=True)
        var = jnp.var(out, axis=(0, 2), keepdims=True)
        out = (out - mu) / jnp.sqrt(var + _BN_EPS)
        out = out * bn_gamma[layer][None, :, None] + bn_beta[layer][None, :, None]
        out = jnp.where(out >= 0, out, _SLOPE * out)
    return out


def kernel(x, edge_index, edge_weights, feature_mask, conv_weights,
           conv_biases, bn_gamma, bn_beta, fc_w, fc_b):
    src, dst = edge_index[0], edge_index[1]
    out = _graph_phase(x, src, dst, edge_weights, conv_weights, conv_biases,
                       bn_gamma, bn_beta)
    x2d = out.reshape(_BATCH, _FC_IN)
    y = jnp.broadcast_to(jax.lax.dynamic_slice(fc_w, (0, 0), (1, _FC_OUT)),
                         (_BATCH, _FC_OUT)) + x2d[:, :_FC_OUT] + fc_b
    return y.reshape(_BATCH, _N_NODES, _DIMS[-1])


# speedup vs baseline: 1.0308x; 1.0308x over previous
import functools

import jax
import jax.numpy as jnp
from jax.experimental import pallas as pl
from jax.experimental.pallas import tpu as pltpu

_N_NODES = 2000
_N_EDGES = 6000
_BATCH = 4
_DIMS = [8, 32, 16, 8, 2]
_HOPS = [3, 3, 3]
_SLOPE = 0.01
_BN_EPS = 1e-5

_FC_IN = _DIMS[-2] * _N_NODES
_FC_OUT = _DIMS[-1] * _N_NODES

_KT = 3200
_NT = 512
_FC_OUT_PAD = 4096


def _fc_body(x_ref, w_ref, b_ref, m_ref, o_ref):
    k = pl.program_id(0)
    n = pl.program_id(1)
    nk = pl.num_programs(0)
    nn = pl.num_programs(1)

    @pl.when(k == 0)
    def _init():
        o_ref[:, pl.ds(n * _NT, _NT)] = jnp.zeros((_BATCH, _NT), jnp.float32)

    o_ref[:, pl.ds(n * _NT, _NT)] += jnp.dot(
        x_ref[...], w_ref[...], preferred_element_type=jnp.float32)

    @pl.when(jnp.logical_and(k == nk - 1, n == nn - 1))
    def _fini():
        o_ref[...] = (o_ref[...] + b_ref[...]) * m_ref[...]


@jax.jit
def _fc_pallas(x2d, fc_w, fc_b, mask_flat):
    pad = _FC_OUT_PAD - _FC_OUT
    b_pad = jnp.broadcast_to(jnp.pad(fc_b, (0, pad))[None, :],
                             (_BATCH, _FC_OUT_PAD))
    m_pad = jnp.broadcast_to(jnp.pad(mask_flat, (0, pad))[None, :],
                             (_BATCH, _FC_OUT_PAD))
    grid = (_FC_IN // _KT, _FC_OUT_PAD // _NT)
    y = pl.pallas_call(
        _fc_body,
        grid=grid,
        in_specs=[
            pl.BlockSpec((_BATCH, _KT), lambda k, n: (0, k)),
            pl.BlockSpec((_KT, _NT), lambda k, n: (k, n)),
            pl.BlockSpec((_BATCH, _FC_OUT_PAD), lambda k, n: (0, 0)),
            pl.BlockSpec((_BATCH, _FC_OUT_PAD), lambda k, n: (0, 0)),
        ],
        out_specs=pl.BlockSpec((_BATCH, _FC_OUT_PAD), lambda k, n: (0, 0)),
        out_shape=jax.ShapeDtypeStruct((_BATCH, _FC_OUT_PAD), jnp.float32),
        compiler_params=pltpu.CompilerParams(
            dimension_semantics=("arbitrary", "arbitrary"),
        ),
    )(x2d, fc_w, b_pad, m_pad)
    return y[:, :_FC_OUT]


def _graph_phase(x, src, dst, ew, conv_weights, conv_biases, bn_gamma, bn_beta):
    deg = jnp.zeros((_N_NODES,), jnp.float32).at[dst].add(ew)
    dinv = jnp.where(deg > 0, 1.0 / jnp.sqrt(deg), 0.0)
    norm = dinv[src] * ew * dinv[dst]

    out = x
    for layer in range(len(_HOPS)):
        h = out
        acc = h @ conv_weights[layer][0]
        for w_hop in conv_weights[layer][1:]:
            msgs = h[:, src, :] * norm[None, :, None]
            h = jnp.zeros_like(h).at[:, dst, :].add(msgs)
            acc = acc + h @ w_hop
        out = acc + conv_biases[layer]
        mu = jnp.mean(out, axis=(0, 2), keepdims=True)
        var = jnp.var(out, axis=(0, 2), keepdims=True)
        out = (out - mu) / jnp.sqrt(var + _BN_EPS)
        out = out * bn_gamma[layer][None, :, None] + bn_beta[layer][None, :, None]
        out = jnp.where(out >= 0, out, _SLOPE * out)
    return out


def kernel(x, edge_index, edge_weights, feature_mask, conv_weights,
           conv_biases, bn_gamma, bn_beta, fc_w, fc_b):
    src, dst = edge_index[0], edge_index[1]
    out = _graph_phase(x, src, dst, edge_weights, conv_weights, conv_biases,
                       bn_gamma, bn_beta)
    x2d = out.reshape(_BATCH, _FC_IN)
    y = _fc_pallas(x2d, fc_w, fc_b, feature_mask.reshape(-1))
    return y.reshape(_BATCH, _N_NODES, _DIMS[-1])


# speedup vs baseline: 12.4341x; 12.0628x over previous
import functools

import jax
import jax.numpy as jnp
from jax import lax
from jax.experimental import pallas as pl
from jax.experimental.pallas import tpu as pltpu
from jax.experimental.pallas import tpu_sc as plsc

_N_NODES = 2000
_N_EDGES = 6000
_BATCH = 4
_DIMS = [8, 32, 16, 8, 2]
_HOPS = [3, 3, 3]
_SLOPE = 0.01
_BN_EPS = 1e-5

_FC_IN = _DIMS[-2] * _N_NODES
_FC_OUT = _DIMS[-1] * _N_NODES

_KT = 3200
_NT = 512
_FC_OUT_PAD = 4096


def _fc_body(x_ref, w_ref, b_ref, m_ref, o_ref):
    k = pl.program_id(0)
    n = pl.program_id(1)
    nk = pl.num_programs(0)
    nn = pl.num_programs(1)

    @pl.when(k == 0)
    def _init():
        o_ref[:, pl.ds(n * _NT, _NT)] = jnp.zeros((_BATCH, _NT), jnp.float32)

    o_ref[:, pl.ds(n * _NT, _NT)] += jnp.dot(
        x_ref[...], w_ref[...], preferred_element_type=jnp.float32)

    @pl.when(jnp.logical_and(k == nk - 1, n == nn - 1))
    def _fini():
        o_ref[...] = (o_ref[...] + b_ref[...]) * m_ref[...]


@jax.jit
def _fc_pallas(x2d, fc_w, fc_b, mask_flat):
    pad = _FC_OUT_PAD - _FC_OUT
    b_pad = jnp.broadcast_to(jnp.pad(fc_b, (0, pad))[None, :],
                             (_BATCH, _FC_OUT_PAD))
    m_pad = jnp.broadcast_to(jnp.pad(mask_flat, (0, pad))[None, :],
                             (_BATCH, _FC_OUT_PAD))
    grid = (_FC_IN // _KT, _FC_OUT_PAD // _NT)
    y = pl.pallas_call(
        _fc_body,
        grid=grid,
        in_specs=[
            pl.BlockSpec((_BATCH, _KT), lambda k, n: (0, k)),
            pl.BlockSpec((_KT, _NT), lambda k, n: (k, n)),
            pl.BlockSpec((_BATCH, _FC_OUT_PAD), lambda k, n: (0, 0)),
            pl.BlockSpec((_BATCH, _FC_OUT_PAD), lambda k, n: (0, 0)),
        ],
        out_specs=pl.BlockSpec((_BATCH, _FC_OUT_PAD), lambda k, n: (0, 0)),
        out_shape=jax.ShapeDtypeStruct((_BATCH, _FC_OUT_PAD), jnp.float32),
        compiler_params=pltpu.CompilerParams(
            dimension_semantics=("arbitrary", "arbitrary"),
        ),
    )(x2d, fc_w, b_pad, m_pad)
    return y[:, :_FC_OUT]


_ROWS_PER_TILE = 64
_A_ROWS = 32 * _ROWS_PER_TILE
_ECHUNK = 400


def _build_a0_sc():
    mesh = plsc.VectorSubcoreMesh(core_axis_name="c", subcore_axis_name="s")

    blk_words = _ROWS_PER_TILE * _N_NODES

    @functools.partial(
        pl.kernel,
        out_type=jax.ShapeDtypeStruct((_A_ROWS * _N_NODES,), jnp.float32),
        mesh=mesh,
        compiler_params=pltpu.CompilerParams(needs_layout_passes=False),
        scratch_types=[
            pltpu.VMEM((blk_words,), jnp.float32),
            pltpu.VMEM((_ECHUNK,), jnp.int32),
            pltpu.VMEM((_ECHUNK,), jnp.int32),
            pltpu.VMEM((_ECHUNK,), jnp.float32),
        ],
    )
    def a0_kernel(src_hbm, dst_hbm, ew_hbm, a_hbm, ablk, srcv, dstv, ewv):
        wid = lax.axis_index("s") * 2 + lax.axis_index("c")
        base = wid * _ROWS_PER_TILE

        def zero_vec(c, _):
            ablk[pl.ds(c * 16, 16)] = jnp.zeros((16,), jnp.float32)
            return 0

        lax.fori_loop(0, blk_words // 16, zero_vec, 0)

        lanes = lax.iota(jnp.int32, 16)

        for chunk in range(_N_EDGES // _ECHUNK):
            off = chunk * _ECHUNK
            pltpu.sync_copy(src_hbm.at[pl.ds(off, _ECHUNK)], srcv)
            pltpu.sync_copy(dst_hbm.at[pl.ds(off, _ECHUNK)], dstv)
            pltpu.sync_copy(ew_hbm.at[pl.ds(off, _ECHUNK)], ewv)

            def scan_group(g, _):
                goff = g * 16
                src16 = srcv[pl.ds(goff, 16)]
                r16 = dstv[pl.ds(goff, 16)] - base
                ew16 = ewv[pl.ds(goff, 16)]
                flat16 = r16 * _N_NODES + src16
                m_in = jnp.logical_and(r16 >= 0, r16 < _ROWS_PER_TILE)
                for e in range(16):
                    m_e = jnp.logical_and(m_in, lanes == e)
                    old = plsc.load_gather(ablk, [flat16], mask=m_e)
                    plsc.store_scatter(ablk, [flat16], old + ew16, mask=m_e)
                return 0

            lax.fori_loop(0, _ECHUNK // 16, scan_group, 0)

        pltpu.sync_copy(ablk, a_hbm.at[pl.ds(base * _N_NODES, blk_words)])

    return a0_kernel


_A0_SC = _build_a0_sc()


def _leaky(v):
    return jnp.where(v >= 0, v, _SLOPE * v)


def _bn_rows(o, gamma_col, beta_col):
    mu = jnp.mean(o, axis=1, keepdims=True)
    var = jnp.mean((o - mu) ** 2, axis=1, keepdims=True)
    return (o - mu) * jax.lax.rsqrt(var + _BN_EPS) * gamma_col + beta_col


def _graph_body(a_ref, h0_ref, m1_ref, m2_ref, m3_ref,
                b1_ref, b2_ref, b3_ref,
                g1_ref, t1_ref, g2_ref, t2_ref, g3_ref, t3_ref, out_ref):
    a = a_ref[...]
    deg = jnp.sum(a, axis=1, keepdims=True)
    dinv = jnp.where(deg > 0, jax.lax.rsqrt(deg), 0.0)
    dinv_n = dinv[:_N_NODES]

    def prop(h):
        u = jnp.dot(a, dinv_n * h, preferred_element_type=jnp.float32)
        return (dinv * u)[:_N_NODES]

    s0 = h0_ref[...]
    s1 = prop(s0)
    s2 = prop(s1)
    s3 = prop(s2)
    s = jnp.concatenate([s0, s1, s2, s3], axis=1)
    o = jnp.dot(s, m1_ref[...], preferred_element_type=jnp.float32) + b1_ref[...]
    o = _leaky(_bn_rows(o, g1_ref[...], t1_ref[...]))

    g = jnp.dot(o, m2_ref[...], preferred_element_type=jnp.float32)
    acc = g[:, 192:256]
    acc = g[:, 128:192] + prop(acc)
    acc = g[:, 64:128] + prop(acc)
    acc = g[:, 0:64] + prop(acc)
    o = acc + b2_ref[...]
    o = _leaky(_bn_rows(o, g2_ref[...], t2_ref[...]))

    g = jnp.dot(o, m3_ref[...], preferred_element_type=jnp.float32)
    acc = g[:, 96:128]
    acc = g[:, 64:96] + prop(acc)
    acc = g[:, 32:64] + prop(acc)
    acc = g[:, 0:32] + prop(acc)
    o = acc + b3_ref[...]
    o = _leaky(_bn_rows(o, g3_ref[...], t3_ref[...]))
    out_ref[...] = o


@jax.jit
def _graph_pallas(a0, h0, conv_weights, conv_biases, bn_gamma, bn_beta):
    eye_b = jnp.eye(_BATCH, dtype=jnp.float32)
    m1 = jnp.concatenate(
        [jnp.kron(eye_b, w) for w in conv_weights[0]], axis=0)
    m2 = jnp.concatenate(
        [jnp.kron(eye_b, w) for w in conv_weights[1]], axis=1)
    m3 = jnp.concatenate(
        [jnp.kron(eye_b, w) for w in conv_weights[2]], axis=1)
    b1 = jnp.tile(conv_biases[0], _BATCH)[None, :]
    b2 = jnp.tile(conv_biases[1], _BATCH)[None, :]
    b3 = jnp.tile(conv_biases[2], _BATCH)[None, :]
    cols = [bn_gamma[0][:, None], bn_beta[0][:, None],
            bn_gamma[1][:, None], bn_beta[1][:, None],
            bn_gamma[2][:, None], bn_beta[2][:, None]]
    return pl.pallas_call(
        _graph_body,
        out_shape=jax.ShapeDtypeStruct((_N_NODES, _BATCH * _DIMS[3]),
                                       jnp.float32),
    )(a0, h0, m1, m2, m3, b1, b2, b3, *cols)


def kernel(x, edge_index, edge_weights, feature_mask, conv_weights,
           conv_biases, bn_gamma, bn_beta, fc_w, fc_b):
    a0 = _A0_SC(edge_index[0], edge_index[1], edge_weights).reshape(
        _A_ROWS, _N_NODES)
    h0 = x.transpose(1, 0, 2).reshape(_N_NODES, _BATCH * _DIMS[0])
    out3 = _graph_pallas(a0, h0, conv_weights, conv_biases, bn_gamma, bn_beta)
    x2d = out3.reshape(_N_NODES, _BATCH, _DIMS[3]).transpose(1, 0, 2).reshape(
        _BATCH, _FC_IN)
    y = _fc_pallas(x2d, fc_w, fc_b, feature_mask.reshape(-1))
    return y.reshape(_BATCH, _N_NODES, _DIMS[-1])


# speedup vs baseline: 12.5512x; 1.0094x over previous
import functools

import jax
import jax.numpy as jnp
from jax import lax
from jax.experimental import pallas as pl
from jax.experimental.pallas import tpu as pltpu
from jax.experimental.pallas import tpu_sc as plsc

_N_NODES = 2000
_N_EDGES = 6000
_BATCH = 4
_DIMS = [8, 32, 16, 8, 2]
_HOPS = [3, 3, 3]
_SLOPE = 0.01
_BN_EPS = 1e-5

_FC_IN = _DIMS[-2] * _N_NODES
_FC_OUT = _DIMS[-1] * _N_NODES

_KT = 3200
_NT = 512
_FC_OUT_PAD = 4096


def _fc_body(x_ref, w_ref, b_ref, m_ref, o_ref):
    k = pl.program_id(0)
    n = pl.program_id(1)
    nk = pl.num_programs(0)
    nn = pl.num_programs(1)

    @pl.when(k == 0)
    def _init():
        o_ref[:, pl.ds(n * _NT, _NT)] = jnp.zeros((_BATCH, _NT), jnp.float32)

    o_ref[:, pl.ds(n * _NT, _NT)] += jnp.dot(
        x_ref[...], w_ref[...], preferred_element_type=jnp.float32)

    @pl.when(jnp.logical_and(k == nk - 1, n == nn - 1))
    def _fini():
        o_ref[...] = (o_ref[...] + b_ref[...]) * m_ref[...]


@jax.jit
def _fc_pallas(x2d, fc_w, fc_b, mask_flat):
    pad = _FC_OUT_PAD - _FC_OUT
    b_pad = jnp.broadcast_to(jnp.pad(fc_b, (0, pad))[None, :],
                             (_BATCH, _FC_OUT_PAD))
    m_pad = jnp.broadcast_to(jnp.pad(mask_flat, (0, pad))[None, :],
                             (_BATCH, _FC_OUT_PAD))
    grid = (_FC_IN // _KT, _FC_OUT_PAD // _NT)
    y = pl.pallas_call(
        _fc_body,
        grid=grid,
        in_specs=[
            pl.BlockSpec((_BATCH, _KT), lambda k, n: (0, k)),
            pl.BlockSpec((_KT, _NT), lambda k, n: (k, n)),
            pl.BlockSpec((_BATCH, _FC_OUT_PAD), lambda k, n: (0, 0)),
            pl.BlockSpec((_BATCH, _FC_OUT_PAD), lambda k, n: (0, 0)),
        ],
        out_specs=pl.BlockSpec((_BATCH, _FC_OUT_PAD), lambda k, n: (0, 0)),
        out_shape=jax.ShapeDtypeStruct((_BATCH, _FC_OUT_PAD), jnp.float32),
        compiler_params=pltpu.CompilerParams(
            dimension_semantics=("arbitrary", "arbitrary"),
        ),
    )(x2d, fc_w, b_pad, m_pad)
    return y[:, :_FC_OUT]


_ROWS_PER_TILE = 64
_A_ROWS = 32 * _ROWS_PER_TILE
_ECHUNK = 400


def _build_a0_sc():
    mesh = plsc.VectorSubcoreMesh(core_axis_name="c", subcore_axis_name="s")

    blk_words = _ROWS_PER_TILE * _N_NODES
    n_chunks = _N_EDGES // _ECHUNK

    @functools.partial(
        pl.kernel,
        out_type=jax.ShapeDtypeStruct((_A_ROWS * _N_NODES,), jnp.float32),
        mesh=mesh,
        compiler_params=pltpu.CompilerParams(needs_layout_passes=False),
        scratch_types=[
            pltpu.VMEM((blk_words,), jnp.float32),
            pltpu.VMEM((_ECHUNK,), jnp.int32),
            pltpu.VMEM((_ECHUNK,), jnp.int32),
            pltpu.VMEM((_ECHUNK,), jnp.float32),
            pltpu.VMEM((_ECHUNK,), jnp.int32),
            pltpu.VMEM((_ECHUNK,), jnp.int32),
            pltpu.VMEM((_ECHUNK,), jnp.float32),
            pltpu.SemaphoreType.DMA,
            pltpu.SemaphoreType.DMA,
            pltpu.SemaphoreType.DMA,
            pltpu.SemaphoreType.DMA,
            pltpu.SemaphoreType.DMA,
            pltpu.SemaphoreType.DMA,
        ],
    )
    def a0_kernel(src_hbm, dst_hbm, ew_hbm, a_hbm, ablk,
                  srcv0, dstv0, ewv0, srcv1, dstv1, ewv1, *sems):
        wid = lax.axis_index("s") * 2 + lax.axis_index("c")
        base = wid * _ROWS_PER_TILE
        bufs = ((srcv0, dstv0, ewv0), (srcv1, dstv1, ewv1))

        def issue(c):
            buf = c % 2
            off = c * _ECHUNK
            sv, dv, ev = bufs[buf]
            return (
                pltpu.async_copy(src_hbm.at[pl.ds(off, _ECHUNK)],
                                 sv, sems[buf * 3 + 0]),
                pltpu.async_copy(dst_hbm.at[pl.ds(off, _ECHUNK)],
                                 dv, sems[buf * 3 + 1]),
                pltpu.async_copy(ew_hbm.at[pl.ds(off, _ECHUNK)],
                                 ev, sems[buf * 3 + 2]),
            )

        pending = issue(0)

        def zero_vec(c, _):
            for j in range(8):
                ablk[pl.ds(c * 128 + j * 16, 16)] = jnp.zeros((16,),
                                                              jnp.float32)
            return 0

        lax.fori_loop(0, blk_words // 128, zero_vec, 0)

        lanes = lax.iota(jnp.int32, 16)

        for chunk in range(n_chunks):
            sv, dv, ev = bufs[chunk % 2]
            for h in pending:
                h.wait()
            if chunk + 1 < n_chunks:
                pending = issue(chunk + 1)

            def scan_group(g, _):
                goff = g * 16
                src16 = sv[pl.ds(goff, 16)]
                r16 = dv[pl.ds(goff, 16)] - base
                ew16 = ev[pl.ds(goff, 16)]
                m_in = jnp.logical_and(r16 >= 0, r16 < _ROWS_PER_TILE)
                cnt = plsc.all_reduce_population_count(m_in)

                @pl.when(cnt[0] > 0)
                def _():
                    flat16 = r16 * _N_NODES + src16

                    def peel(_, m32):
                        m = m32 > 0
                        e = plsc.all_reduce_ffs(m)
                        m_e = jnp.logical_and(m, lanes == e)
                        old = plsc.load_gather(ablk, [flat16], mask=m_e)
                        plsc.store_scatter(ablk, [flat16], old + ew16,
                                           mask=m_e)
                        return jnp.where(m_e, 0, m32)

                    lax.fori_loop(0, cnt[0], peel, m_in.astype(jnp.int32))
                return 0

            lax.fori_loop(0, _ECHUNK // 16, scan_group, 0)

        pltpu.sync_copy(ablk, a_hbm.at[pl.ds(base * _N_NODES, blk_words)])

    return a0_kernel


_A0_SC = _build_a0_sc()


def _leaky(v):
    return jnp.where(v >= 0, v, _SLOPE * v)


def _bn_rows(o, gamma_col, beta_col):
    mu = jnp.mean(o, axis=1, keepdims=True)
    var = jnp.mean((o - mu) ** 2, axis=1, keepdims=True)
    return (o - mu) * jax.lax.rsqrt(var + _BN_EPS) * gamma_col + beta_col


def _graph_body(a_ref, h0_ref, m1_ref, m2_ref, m3_ref,
                b1_ref, b2_ref, b3_ref,
                g1_ref, t1_ref, g2_ref, t2_ref, g3_ref, t3_ref, out_ref):
    a = a_ref[...]
    deg = jnp.sum(a, axis=1, keepdims=True)
    dinv = jnp.where(deg > 0, jax.lax.rsqrt(deg), 0.0)
    dinv_n = dinv[:_N_NODES]

    def prop(h):
        u = jnp.dot(a, dinv_n * h, preferred_element_type=jnp.float32)
        return (dinv * u)[:_N_NODES]

    s0 = h0_ref[...]
    s1 = prop(s0)
    s2 = prop(s1)
    s3 = prop(s2)
    s = jnp.concatenate([s0, s1, s2, s3], axis=1)
    o = jnp.dot(s, m1_ref[...], preferred_element_type=jnp.float32) + b1_ref[...]
    o = _leaky(_bn_rows(o, g1_ref[...], t1_ref[...]))

    g = jnp.dot(o, m2_ref[...], preferred_element_type=jnp.float32)
    acc = g[:, 192:256]
    acc = g[:, 128:192] + prop(acc)
    acc = g[:, 64:128] + prop(acc)
    acc = g[:, 0:64] + prop(acc)
    o = acc + b2_ref[...]
    o = _leaky(_bn_rows(o, g2_ref[...], t2_ref[...]))

    g = jnp.dot(o, m3_ref[...], preferred_element_type=jnp.float32)
    acc = g[:, 96:128]
    acc = g[:, 64:96] + prop(acc)
    acc = g[:, 32:64] + prop(acc)
    acc = g[:, 0:32] + prop(acc)
    o = acc + b3_ref[...]
    o = _leaky(_bn_rows(o, g3_ref[...], t3_ref[...]))
    out_ref[...] = o


@jax.jit
def _graph_pallas(a0, h0, conv_weights, conv_biases, bn_gamma, bn_beta):
    eye_b = jnp.eye(_BATCH, dtype=jnp.float32)
    m1 = jnp.concatenate(
        [jnp.kron(eye_b, w) for w in conv_weights[0]], axis=0)
    m2 = jnp.concatenate(
        [jnp.kron(eye_b, w) for w in conv_weights[1]], axis=1)
    m3 = jnp.concatenate(
        [jnp.kron(eye_b, w) for w in conv_weights[2]], axis=1)
    b1 = jnp.tile(conv_biases[0], _BATCH)[None, :]
    b2 = jnp.tile(conv_biases[1], _BATCH)[None, :]
    b3 = jnp.tile(conv_biases[2], _BATCH)[None, :]
    cols = [bn_gamma[0][:, None], bn_beta[0][:, None],
            bn_gamma[1][:, None], bn_beta[1][:, None],
            bn_gamma[2][:, None], bn_beta[2][:, None]]
    return pl.pallas_call(
        _graph_body,
        out_shape=jax.ShapeDtypeStruct((_N_NODES, _BATCH * _DIMS[3]),
                                       jnp.float32),
    )(a0, h0, m1, m2, m3, b1, b2, b3, *cols)


def kernel(x, edge_index, edge_weights, feature_mask, conv_weights,
           conv_biases, bn_gamma, bn_beta, fc_w, fc_b):
    a0 = _A0_SC(edge_index[0], edge_index[1], edge_weights).reshape(
        _A_ROWS, _N_NODES)
    h0 = x.transpose(1, 0, 2).reshape(_N_NODES, _BATCH * _DIMS[0])
    out3 = _graph_pallas(a0, h0, conv_weights, conv_biases, bn_gamma, bn_beta)
    x2d = out3.reshape(_N_NODES, _BATCH, _DIMS[3]).transpose(1, 0, 2).reshape(
        _BATCH, _FC_IN)
    y = _fc_pallas(x2d, fc_w, fc_b, feature_mask.reshape(-1))
    return y.reshape(_BATCH, _N_NODES, _DIMS[-1])
